# Initial kernel scaffold; baseline (speedup 1.0000x reference)
import jax
import jax.numpy as jnp
from jax.experimental import pallas as pl


def kernel(hidden_states, gate_w, experts_w1, experts_w2):
    raise NotImplementedError("write your pallas kernel here")


# speedup vs baseline: 1.2652x; 1.2652x over previous
import jax
import jax.numpy as jnp
from jax.experimental import pallas as pl
from jax.experimental.pallas import tpu as pltpu

T = 2048
H = 1024
F = 2048
E = 8
K = 2

NEG_INF = float("-inf")


def _router_kernel(x_ref, gwt_ref, w_ref):
    logits = jnp.dot(x_ref[...], gwt_ref[...], preferred_element_type=jnp.float32)
    m = jnp.max(logits, axis=-1, keepdims=True)
    p = jnp.exp(logits - m)
    p = p / jnp.sum(p, axis=-1, keepdims=True)
    idx = jax.lax.broadcasted_iota(jnp.int32, (T, E), 1)
    m1 = jnp.max(p, axis=-1, keepdims=True)
    i1 = jnp.min(jnp.where(p == m1, idx, E), axis=-1, keepdims=True)
    one1 = idx == i1
    p2 = jnp.where(one1, NEG_INF, p)
    m2 = jnp.max(p2, axis=-1, keepdims=True)
    i2 = jnp.min(jnp.where(p2 == m2, idx, E), axis=-1, keepdims=True)
    one2 = idx == i2
    denom = m1 + m2
    w_ref[...] = jnp.where(one1, m1 / denom, 0.0) + jnp.where(one2, m2 / denom, 0.0)


def _moe_kernel(x_ref, w1_ref, w2_ref, wfull_ref, out_ref):
    e = pl.program_id(1)
    h = jnp.dot(x_ref[...], w1_ref[0], preferred_element_type=jnp.float32)
    h = h * jax.nn.sigmoid(h)
    y = jnp.dot(h, w2_ref[0], preferred_element_type=jnp.float32)
    ids = jax.lax.broadcasted_iota(jnp.int32, (1, E), 1)
    wcol = jnp.sum(wfull_ref[...] * (ids == e).astype(jnp.float32), axis=1,
                   keepdims=True)
    contrib = wcol * y

    @pl.when(e == 0)
    def _():
        out_ref[...] = contrib

    @pl.when(e > 0)
    def _():
        out_ref[...] += contrib


def kernel(hidden_states, gate_w, experts_w1, experts_w2):
    wfull = pl.pallas_call(
        _router_kernel,
        out_shape=jax.ShapeDtypeStruct((T, E), jnp.float32),
    )(hidden_states, gate_w.T)

    BT = 512
    NT = T // BT
    out = pl.pallas_call(
        _moe_kernel,
        grid=(NT, E),
        in_specs=[
            pl.BlockSpec((BT, H), lambda t, e: (t, 0)),
            pl.BlockSpec((1, H, F), lambda t, e: (e, 0, 0)),
            pl.BlockSpec((1, F, H), lambda t, e: (e, 0, 0)),
            pl.BlockSpec((BT, E), lambda t, e: (t, 0)),
        ],
        out_specs=pl.BlockSpec((BT, H), lambda t, e: (t, 0)),
        out_shape=jax.ShapeDtypeStruct((T, H), jnp.float32),
    )(hidden_states, experts_w1, experts_w2, wfull)
    return out
